# baseline (speedup 1.0000x reference)
import functools

import jax
import jax.numpy as jnp
from jax import lax
from jax.experimental import pallas as pl
from jax.experimental.pallas import tpu as pltpu
import jax.experimental.pallas.tpu_sc as plsc

NC = 2
NS = 16
LANES = 16
K = 400
KG = 40


def _elu(x):
    return jnp.where(x > 0, x, jnp.exp(jnp.minimum(x, 0.0)) - 1.0)


def _dot(x, w):
    return jnp.dot(x.astype(jnp.bfloat16), w.astype(jnp.bfloat16),
                   preferred_element_type=jnp.float32)


def _pack(x):
    r, w = x.shape
    return jax.lax.bitcast_convert_type(
        x.astype(jnp.bfloat16).reshape(r, w // 2, 2), jnp.int32)


def _unpack(x):
    r, w2 = x.shape
    b = jax.lax.bitcast_convert_type(x, jnp.bfloat16)
    return b.astype(jnp.float32).reshape(r, 2 * w2)


def _tc1_body(x_ref, gf_ref, w1_ref, b1_ref, w2_ref, b2_ref, w21_ref,
              b21_ref, a_ref, b_ref):
    api = x_ref.shape[1]
    w1 = w1_ref[...]
    b1v = jnp.dot(gf_ref[...], w1[api:], preferred_element_type=jnp.float32) + b1_ref[...]
    h = _elu(_dot(x_ref[...], w1[:api]) + b1v)
    out = _elu(_dot(h, w2_ref[...]) + b2_ref[...])
    w21 = w21_ref[...]
    a_ref[...] = _dot(out, w21[:api]) + b21_ref[...]
    b_ref[...] = _dot(out, w21[api:])


def _tc2_body(sa_ref, sb_ref, w22_ref, b22_ref, w4e_ref, b4_ref, skip_ref,
              t_ref):
    h = _elu(sa_ref[...] + sb_ref[...])
    re = h.shape[0]
    sk = _elu(_dot(h, w22_ref[...]) + b22_ref[...])
    skip_ref[...] = jnp.transpose(sk.reshape(re, LANES, LANES),
                                  (1, 0, 2)).reshape(LANES, re * LANES)
    t_ref[...] = (_dot(sk, w4e_ref[...]) + b4_ref[...]).astype(jnp.bfloat16)


def _tc3_body(f0_ref, w31_ref, b31_ref, w32_ref, b32_ref, w4sd_ref,
              feats_ref, c_ref, d_ref):
    rn = f0_ref.shape[1] // LANES
    api = LANES * LANES
    f0 = jnp.transpose(f0_ref[...].reshape(LANES, rn, LANES),
                       (1, 0, 2)).reshape(rn, api)
    h = _elu(_dot(f0, w31_ref[...]) + b31_ref[...])
    f = _elu(_dot(h, w32_ref[...]) + b32_ref[...])
    feats_ref[...] = f
    w = w4sd_ref[...]
    c_ref[...] = _dot(f, w[:api])
    d_ref[...] = _dot(f, w[api:])


def _tc4_body(ce_ref, de_ref, t_ref, w42_ref, b42_ref, fcwt_ref, fcb_ref,
              ew_ref):
    h4 = _elu(ce_ref[...] + de_ref[...] + t_ref[...].astype(jnp.float32))
    ef3 = _elu(_dot(h4, w42_ref[...]) + b42_ref[...])
    v = jnp.sum(ef3 * fcwt_ref[...], axis=1, keepdims=True) + fcb_ref[...]
    ew_ref[...] = jax.nn.sigmoid(v)


def _row_spec(rows, cols):
    return pl.BlockSpec((rows, cols), lambda i: (i, 0))


def _full_spec(rows, cols):
    return pl.BlockSpec((rows, cols), lambda i: (0, 0))


def kernel(node_feats, graph_feats, m1_w1, m1_b1, m1_w2, m1_b2,
           m2_w1, m2_b1, m2_w2, m2_b2, m3_w1, m3_b1, m3_w2, m3_b2,
           m4_w1, m4_b1, m4_w2, m4_b2, fc_w, fc_b, edge_index):
    n, api = node_feats.shape
    hid = m1_w1.shape[1]
    e = edge_index.shape[1]
    mash = graph_feats.shape[0]
    f32 = jnp.float32
    bf16 = jnp.bfloat16

    src = edge_index[0]
    dst = edge_index[1]

    rn = 1000
    re = 1000
    assert n % rn == 0 and e % re == 0 and e % K == 0 and e % KG == 0 and api == LANES * LANES
    nw = NC * NS
    n_chunks = e // K
    half = n // NC
    tbl_rows = half + LANES

    a_mat, b_mat = pl.pallas_call(
        _tc1_body,
        grid=(n // rn,),
        in_specs=[
            _row_spec(rn, api),
            _full_spec(1, mash),
            _full_spec(api + mash, hid),
            _full_spec(1, hid),
            _full_spec(hid, api),
            _full_spec(1, api),
            _full_spec(2 * api, hid),
            _full_spec(1, hid),
        ],
        out_specs=[_row_spec(rn, hid), _row_spec(rn, hid)],
        out_shape=[jax.ShapeDtypeStruct((n, hid), f32),
                   jax.ShapeDtypeStruct((n, hid), f32)],
    )(node_feats, graph_feats.reshape(1, mash), m1_w1, m1_b1.reshape(1, hid),
      m1_w2, m1_b2.reshape(1, api), m2_w1, m2_b1.reshape(1, hid))

    mesh = plsc.VectorSubcoreMesh(core_axis_name="c", subcore_axis_name="s",
                                  num_cores=NC, num_subcores=NS)
    ng_chunks = e // KG
    steps_g = (ng_chunks + nw - 1) // nw

    def _gather2(a_hbm, b_hbm, src_hbm, dst_hbm, sa_hbm, sb_hbm,
                 sidx0, didx0, sidx1, didx1, ra0, rb0, ra1, rb1,
                 sa0, sb0, sa1, sb1):
        wid = lax.axis_index("s") * NC + lax.axis_index("c")

        def issue(chunk, sidx_v, didx_v, rows_a, rows_b, sema, semb):
            off = chunk * KG
            pltpu.sync_copy(src_hbm.at[pl.ds(off, KG)], sidx_v)
            pltpu.sync_copy(dst_hbm.at[pl.ds(off, KG)], didx_v)
            pltpu.async_copy(a_hbm.at[sidx_v], rows_a, sema)
            pltpu.async_copy(b_hbm.at[didx_v], rows_b, semb)

        def drain_write(chunk, rows_a, rows_b, sema, semb):
            off = chunk * KG
            pltpu.make_async_copy(a_hbm.at[pl.ds(0, KG)], rows_a, sema).wait()
            pltpu.make_async_copy(b_hbm.at[pl.ds(0, KG)], rows_b, semb).wait()
            pltpu.sync_copy(rows_a, sa_hbm.at[pl.ds(off, KG)])
            pltpu.sync_copy(rows_b, sb_hbm.at[pl.ds(off, KG)])

        ch0 = wid

        @pl.when(ch0 < ng_chunks)
        def _():
            issue(ch0, sidx0, didx0, ra0, rb0, sa0, sb0)

        g_pairs = (steps_g + 1) // 2

        def pair(pi, carry):
            it0 = 2 * pi
            cA = wid + it0 * nw
            cB = cA + nw
            cA2 = cB + nw

            @pl.when(cB < ng_chunks)
            def _():
                issue(cB, sidx1, didx1, ra1, rb1, sa1, sb1)

            @pl.when(cA < ng_chunks)
            def _():
                drain_write(cA, ra0, rb0, sa0, sb0)

            @pl.when(cA2 < ng_chunks)
            def _():
                issue(cA2, sidx0, didx0, ra0, rb0, sa0, sb0)

            @pl.when(cB < ng_chunks)
            def _():
                drain_write(cB, ra1, rb1, sa1, sb1)

            return carry

        lax.fori_loop(0, g_pairs, pair, 0)

    _gather2_k = functools.partial(
        pl.kernel,
        out_type=(jax.ShapeDtypeStruct((e, hid), f32),
                  jax.ShapeDtypeStruct((e, hid), f32)),
        mesh=mesh,
        scratch_types=[
            pltpu.VMEM((KG,), jnp.int32),
            pltpu.VMEM((KG,), jnp.int32),
            pltpu.VMEM((KG,), jnp.int32),
            pltpu.VMEM((KG,), jnp.int32),
            pltpu.VMEM((KG, hid), f32),
            pltpu.VMEM((KG, hid), f32),
            pltpu.VMEM((KG, hid), f32),
            pltpu.VMEM((KG, hid), f32),
            pltpu.SemaphoreType.DMA,
            pltpu.SemaphoreType.DMA,
            pltpu.SemaphoreType.DMA,
            pltpu.SemaphoreType.DMA,
        ],
    )
    sa_mat, sb_mat = _gather2_k(_gather2)(a_mat, b_mat, src, dst)

    ngh_chunks = (e // 2) // KG
    steps_gh = (ngh_chunks + nw - 1) // nw

    def _gather2h(a_hbm, b_hbm, src_hbm, dst_hbm, sa_hbm, sb_hbm,
                  sidx0, didx0, sidx1, didx1, ra0, rb0, ra1, rb1,
                  sa0, sb0, sa1, sb1):
        wid = lax.axis_index("s") * NC + lax.axis_index("c")

        def issue(chunk, sidx_v, didx_v, rows_a, rows_b, sema, semb):
            off = chunk * KG
            pltpu.sync_copy(src_hbm.at[pl.ds(off, KG)], sidx_v)
            pltpu.sync_copy(dst_hbm.at[pl.ds(off, KG)], didx_v)
            pltpu.async_copy(a_hbm.at[sidx_v], rows_a, sema)
            pltpu.async_copy(b_hbm.at[didx_v], rows_b, semb)

        def drain_write(chunk, rows_a, rows_b, sema, semb):
            off = chunk * KG
            pltpu.make_async_copy(a_hbm.at[pl.ds(0, KG)], rows_a, sema).wait()
            pltpu.make_async_copy(b_hbm.at[pl.ds(0, KG)], rows_b, semb).wait()
            pltpu.sync_copy(rows_a, sa_hbm.at[pl.ds(off, KG)])
            pltpu.sync_copy(rows_b, sb_hbm.at[pl.ds(off, KG)])

        @pl.when(wid < ngh_chunks)
        def _():
            issue(wid, sidx0, didx0, ra0, rb0, sa0, sb0)

        g_pairs = (steps_gh + 1) // 2

        def pair(pi, carry):
            it0 = 2 * pi
            cA = wid + it0 * nw
            cB = cA + nw
            cA2 = cB + nw

            @pl.when(cB < ngh_chunks)
            def _():
                issue(cB, sidx1, didx1, ra1, rb1, sa1, sb1)

            @pl.when(cA < ngh_chunks)
            def _():
                drain_write(cA, ra0, rb0, sa0, sb0)

            @pl.when(cA2 < ngh_chunks)
            def _():
                issue(cA2, sidx0, didx0, ra0, rb0, sa0, sb0)

            @pl.when(cB < ngh_chunks)
            def _():
                drain_write(cB, ra1, rb1, sa1, sb1)

            return carry

        lax.fori_loop(0, g_pairs, pair, 0)

    _gather2h_k = functools.partial(
        pl.kernel,
        out_type=(jax.ShapeDtypeStruct((e // 2, hid), f32),
                  jax.ShapeDtypeStruct((e // 2, hid), f32)),
        mesh=mesh,
        scratch_types=[
            pltpu.VMEM((KG,), jnp.int32),
            pltpu.VMEM((KG,), jnp.int32),
            pltpu.VMEM((KG,), jnp.int32),
            pltpu.VMEM((KG,), jnp.int32),
            pltpu.VMEM((KG, hid), f32),
            pltpu.VMEM((KG, hid), f32),
            pltpu.VMEM((KG, hid), f32),
            pltpu.VMEM((KG, hid), f32),
            pltpu.SemaphoreType.DMA,
            pltpu.SemaphoreType.DMA,
            pltpu.SemaphoreType.DMA,
            pltpu.SemaphoreType.DMA,
        ],
    )

    skip_g, t_mat = pl.pallas_call(
        _tc2_body,
        grid=(e // re,),
        in_specs=[
            _row_spec(re, hid),
            _row_spec(re, hid),
            _full_spec(hid, api),
            _full_spec(1, api),
            _full_spec(api, hid),
            _full_spec(1, hid),
        ],
        out_specs=[pl.BlockSpec((LANES, re * LANES), lambda i: (0, i)),
                   _row_spec(re, hid)],
        out_shape=[jax.ShapeDtypeStruct((LANES, e * LANES), f32),
                   jax.ShapeDtypeStruct((e, hid), jnp.bfloat16)],
    )(sa_mat, sb_mat, m2_w2, m2_b2.reshape(1, api), m4_w1[2 * api:],
      m4_b1.reshape(1, hid))


    n_chunks2 = e // K
    pairs = n_chunks2 // 2
    assert n_chunks2 % 2 == 0

    @functools.partial(
        pl.kernel,
        out_type=jax.ShapeDtypeStruct((LANES, n * LANES), f32),
        mesh=mesh,
        scratch_types=[
            pltpu.VMEM((LANES,), jnp.int32),
            pltpu.VMEM((K,), jnp.int32),
            pltpu.VMEM((K,), jnp.int32),
            pltpu.VMEM((K * LANES,), f32),
            pltpu.VMEM((K * LANES,), f32),
            pltpu.VMEM((tbl_rows * LANES,), f32),
            pltpu.SemaphoreType.DMA,
            pltpu.SemaphoreType.DMA,
            pltpu.SemaphoreType.DMA,
            pltpu.SemaphoreType.DMA,
        ],
    )
    def _sc2(skip_hbm, dst_hbm, iota_hbm, out_hbm, iv, didx0, didx1, rows0,
             rows1, tab_v, sd0, sd1, sr0, sr1):
        c = lax.axis_index("c")
        s = lax.axis_index("s")
        base = c * half
        pltpu.sync_copy(iota_hbm.at[pl.ds(0, LANES)], iv)
        tvec = half + iv[pl.ds(0, LANES)]

        def zr(rr, carry):
            tab_v[pl.ds(rr * LANES, LANES)] = jnp.zeros((LANES,), f32)
            return carry

        lax.fori_loop(0, tbl_rows, zr, 0)

        pltpu.async_copy(dst_hbm.at[pl.ds(0, K)], didx0, sd0)
        pltpu.async_copy(skip_hbm.at[s].at[pl.ds(0, K * LANES)], rows0, sr0)
        pltpu.async_copy(dst_hbm.at[pl.ds(K, K)], didx1, sd1)
        pltpu.async_copy(skip_hbm.at[s].at[pl.ds(K * LANES, K * LANES)],
                         rows1, sr1)

        def process(didx_v, rows_v):
            def grp(j, carry2):
                v = didx_v[pl.ds(j * LANES, LANES)]
                m = (v >= base) & (v < base + half)
                lv = jnp.where(m, v - base, tvec) * LANES
                for l in range(LANES):
                    p = lv[l]
                    q = (j * LANES + l) * LANES
                    tab_v[pl.ds(p, LANES)] = (
                        tab_v[pl.ds(p, LANES)] + rows_v[pl.ds(q, LANES)])
                return carry2

            lax.fori_loop(0, K // LANES, grp, 0)

        def pair(pi, carry):
            c0 = 2 * pi

            pltpu.make_async_copy(dst_hbm.at[pl.ds(0, K)], didx0, sd0).wait()
            pltpu.make_async_copy(skip_hbm.at[s].at[pl.ds(0, K * LANES)],
                                  rows0, sr0).wait()
            process(didx0, rows0)

            @pl.when(c0 + 2 < n_chunks2)
            def _():
                off = c0 + 2
                pltpu.async_copy(dst_hbm.at[pl.ds(off * K, K)], didx0, sd0)
                pltpu.async_copy(
                    skip_hbm.at[s].at[pl.ds(off * K * LANES, K * LANES)],
                    rows0, sr0)

            pltpu.make_async_copy(dst_hbm.at[pl.ds(0, K)], didx1, sd1).wait()
            pltpu.make_async_copy(skip_hbm.at[s].at[pl.ds(0, K * LANES)],
                                  rows1, sr1).wait()
            process(didx1, rows1)

            @pl.when(c0 + 3 < n_chunks2)
            def _():
                off = c0 + 3
                pltpu.async_copy(dst_hbm.at[pl.ds(off * K, K)], didx1, sd1)
                pltpu.async_copy(
                    skip_hbm.at[s].at[pl.ds(off * K * LANES, K * LANES)],
                    rows1, sr1)

            return carry

        lax.fori_loop(0, pairs, pair, 0)
        pltpu.sync_copy(tab_v.at[pl.ds(0, half * LANES)],
                        out_hbm.at[s].at[pl.ds(base * LANES, half * LANES)])

    feats0_g = _sc2(skip_g, dst, jnp.arange(LANES, dtype=jnp.int32))

    feats, c_mat, d_mat = pl.pallas_call(
        _tc3_body,
        grid=(n // rn,),
        in_specs=[
            pl.BlockSpec((LANES, rn * LANES), lambda i: (0, i)),
            _full_spec(api, hid),
            _full_spec(1, hid),
            _full_spec(hid, api),
            _full_spec(1, api),
            _full_spec(2 * api, hid),
        ],
        out_specs=[_row_spec(rn, api), _row_spec(rn, hid),
                   _row_spec(rn, hid)],
        out_shape=[jax.ShapeDtypeStruct((n, api), f32),
                   jax.ShapeDtypeStruct((n, hid), f32),
                   jax.ShapeDtypeStruct((n, hid), f32)],
    )(feats0_g, m3_w1, m3_b1.reshape(1, hid), m3_w2, m3_b2.reshape(1, api),
      m4_w1[:2 * api])

    eh = e // 2

    def _tc4_call(ce_h, de_h, t_h):
        return pl.pallas_call(
            _tc4_body,
            grid=(eh // re,),
            in_specs=[
                _row_spec(re, hid),
                _row_spec(re, hid),
                _row_spec(re, hid),
                _full_spec(hid, api),
                _full_spec(1, api),
                _full_spec(1, api),
                _full_spec(1, 1),
            ],
            out_specs=_row_spec(re, 1),
            out_shape=jax.ShapeDtypeStruct((eh, 1), f32),
        )(ce_h, de_h, t_h, m4_w2, m4_b2.reshape(1, api),
          fc_w.reshape(1, api), fc_b.reshape(1, 1))

    ce0, de0 = _gather2h_k(_gather2h)(c_mat, d_mat, src[:eh], dst[:eh])
    ce1, de1 = _gather2h_k(_gather2h)(c_mat, d_mat, src[eh:], dst[eh:])
    ew0 = _tc4_call(ce0, de0, t_mat[:eh])
    ew1 = _tc4_call(ce1, de1, t_mat[eh:])
    ew = jnp.concatenate([ew0, ew1], axis=0)

    return (ew, feats)


# speedup vs baseline: 1.0242x; 1.0242x over previous
import functools

import jax
import jax.numpy as jnp
from jax import lax
from jax.experimental import pallas as pl
from jax.experimental.pallas import tpu as pltpu
import jax.experimental.pallas.tpu_sc as plsc

NC = 2
NS = 16
LANES = 16
K = 400
KG = 40


def _elu(x):
    return jnp.where(x > 0, x, jnp.exp(jnp.minimum(x, 0.0)) - 1.0)


def _dot(x, w):
    return jnp.dot(x.astype(jnp.bfloat16), w.astype(jnp.bfloat16),
                   preferred_element_type=jnp.float32)


def _pack(x):
    r, w = x.shape
    return jax.lax.bitcast_convert_type(
        x.astype(jnp.bfloat16).reshape(r, w // 2, 2), jnp.int32)


def _unpack(x):
    r, w2 = x.shape
    b = jax.lax.bitcast_convert_type(x, jnp.bfloat16)
    return b.astype(jnp.float32).reshape(r, 2 * w2)


def _tc1_body(x_ref, gf_ref, w1_ref, b1_ref, w2_ref, b2_ref, w21_ref,
              b21_ref, a_ref, b_ref):
    api = x_ref.shape[1]
    w1 = w1_ref[...]
    b1v = jnp.dot(gf_ref[...], w1[api:], preferred_element_type=jnp.float32) + b1_ref[...]
    h = _elu(_dot(x_ref[...], w1[:api]) + b1v)
    out = _elu(_dot(h, w2_ref[...]) + b2_ref[...])
    w21 = w21_ref[...]
    a_ref[...] = _dot(out, w21[:api]) + b21_ref[...]
    b_ref[...] = _dot(out, w21[api:])


def _tc2_body(sa_ref, sb_ref, w22_ref, b22_ref, w4e_ref, b4_ref, skip_ref,
              t_ref):
    h = _elu(sa_ref[...] + sb_ref[...])
    re = h.shape[0]
    sk = _elu(_dot(h, w22_ref[...]) + b22_ref[...])
    skip_ref[...] = jnp.transpose(sk.reshape(re, LANES, LANES),
                                  (1, 0, 2)).reshape(LANES, re * LANES)
    t_ref[...] = (_dot(sk, w4e_ref[...]) + b4_ref[...]).astype(jnp.bfloat16)


def _tc3_body(f0_ref, w31_ref, b31_ref, w32_ref, b32_ref, w4sd_ref,
              feats_ref, c_ref, d_ref):
    rn = f0_ref.shape[1] // LANES
    api = LANES * LANES
    f0 = jnp.transpose(f0_ref[...].reshape(LANES, rn, LANES),
                       (1, 0, 2)).reshape(rn, api)
    h = _elu(_dot(f0, w31_ref[...]) + b31_ref[...])
    f = _elu(_dot(h, w32_ref[...]) + b32_ref[...])
    feats_ref[...] = f
    w = w4sd_ref[...]
    c_ref[...] = _dot(f, w[:api])
    d_ref[...] = _dot(f, w[api:])


def _tc4_body(ce_ref, de_ref, t_ref, w42_ref, b42_ref, fcwt_ref, fcb_ref,
              ew_ref):
    h4 = _elu(ce_ref[...] + de_ref[...] + t_ref[...].astype(jnp.float32))
    ef3 = _elu(_dot(h4, w42_ref[...]) + b42_ref[...])
    v = jnp.sum(ef3 * fcwt_ref[...], axis=1, keepdims=True) + fcb_ref[...]
    ew_ref[...] = jax.nn.sigmoid(v)


def _row_spec(rows, cols):
    return pl.BlockSpec((rows, cols), lambda i: (i, 0))


def _full_spec(rows, cols):
    return pl.BlockSpec((rows, cols), lambda i: (0, 0))


def kernel(node_feats, graph_feats, m1_w1, m1_b1, m1_w2, m1_b2,
           m2_w1, m2_b1, m2_w2, m2_b2, m3_w1, m3_b1, m3_w2, m3_b2,
           m4_w1, m4_b1, m4_w2, m4_b2, fc_w, fc_b, edge_index):
    n, api = node_feats.shape
    hid = m1_w1.shape[1]
    e = edge_index.shape[1]
    mash = graph_feats.shape[0]
    f32 = jnp.float32
    bf16 = jnp.bfloat16

    src = edge_index[0]
    dst = edge_index[1]

    rn = 1000
    re = 1000
    assert n % rn == 0 and e % re == 0 and e % K == 0 and e % KG == 0 and api == LANES * LANES
    nw = NC * NS
    n_chunks = e // K
    half = n // NC
    tbl_rows = half + LANES

    a_mat, b_mat = pl.pallas_call(
        _tc1_body,
        grid=(n // rn,),
        in_specs=[
            _row_spec(rn, api),
            _full_spec(1, mash),
            _full_spec(api + mash, hid),
            _full_spec(1, hid),
            _full_spec(hid, api),
            _full_spec(1, api),
            _full_spec(2 * api, hid),
            _full_spec(1, hid),
        ],
        out_specs=[_row_spec(rn, hid), _row_spec(rn, hid)],
        out_shape=[jax.ShapeDtypeStruct((n, hid), f32),
                   jax.ShapeDtypeStruct((n, hid), f32)],
    )(node_feats, graph_feats.reshape(1, mash), m1_w1, m1_b1.reshape(1, hid),
      m1_w2, m1_b2.reshape(1, api), m2_w1, m2_b1.reshape(1, hid))

    mesh = plsc.VectorSubcoreMesh(core_axis_name="c", subcore_axis_name="s",
                                  num_cores=NC, num_subcores=NS)
    ng_chunks = e // KG
    steps_g = (ng_chunks + nw - 1) // nw

    def _gather2(a_hbm, b_hbm, src_hbm, dst_hbm, sa_hbm, sb_hbm,
                 sidx0, didx0, sidx1, didx1, ra0, rb0, ra1, rb1,
                 sa0, sb0, sa1, sb1):
        wid = lax.axis_index("s") * NC + lax.axis_index("c")

        def issue(chunk, sidx_v, didx_v, rows_a, rows_b, sema, semb):
            off = chunk * KG
            pltpu.sync_copy(src_hbm.at[pl.ds(off, KG)], sidx_v)
            pltpu.sync_copy(dst_hbm.at[pl.ds(off, KG)], didx_v)
            pltpu.async_copy(a_hbm.at[sidx_v], rows_a, sema)
            pltpu.async_copy(b_hbm.at[didx_v], rows_b, semb)

        def drain_write(chunk, rows_a, rows_b, sema, semb):
            off = chunk * KG
            pltpu.make_async_copy(a_hbm.at[pl.ds(0, KG)], rows_a, sema).wait()
            pltpu.make_async_copy(b_hbm.at[pl.ds(0, KG)], rows_b, semb).wait()
            pltpu.sync_copy(rows_a, sa_hbm.at[pl.ds(off, KG)])
            pltpu.sync_copy(rows_b, sb_hbm.at[pl.ds(off, KG)])

        ch0 = wid

        @pl.when(ch0 < ng_chunks)
        def _():
            issue(ch0, sidx0, didx0, ra0, rb0, sa0, sb0)

        g_pairs = (steps_g + 1) // 2

        def pair(pi, carry):
            it0 = 2 * pi
            cA = wid + it0 * nw
            cB = cA + nw
            cA2 = cB + nw

            @pl.when(cB < ng_chunks)
            def _():
                issue(cB, sidx1, didx1, ra1, rb1, sa1, sb1)

            @pl.when(cA < ng_chunks)
            def _():
                drain_write(cA, ra0, rb0, sa0, sb0)

            @pl.when(cA2 < ng_chunks)
            def _():
                issue(cA2, sidx0, didx0, ra0, rb0, sa0, sb0)

            @pl.when(cB < ng_chunks)
            def _():
                drain_write(cB, ra1, rb1, sa1, sb1)

            return carry

        lax.fori_loop(0, g_pairs, pair, 0)

    _gather2_k = functools.partial(
        pl.kernel,
        out_type=(jax.ShapeDtypeStruct((e, hid), f32),
                  jax.ShapeDtypeStruct((e, hid), f32)),
        mesh=mesh,
        scratch_types=[
            pltpu.VMEM((KG,), jnp.int32),
            pltpu.VMEM((KG,), jnp.int32),
            pltpu.VMEM((KG,), jnp.int32),
            pltpu.VMEM((KG,), jnp.int32),
            pltpu.VMEM((KG, hid), f32),
            pltpu.VMEM((KG, hid), f32),
            pltpu.VMEM((KG, hid), f32),
            pltpu.VMEM((KG, hid), f32),
            pltpu.SemaphoreType.DMA,
            pltpu.SemaphoreType.DMA,
            pltpu.SemaphoreType.DMA,
            pltpu.SemaphoreType.DMA,
        ],
    )
    sa_mat, sb_mat = _gather2_k(_gather2)(a_mat, b_mat, src, dst)

    skip_g, t_mat = pl.pallas_call(
        _tc2_body,
        grid=(e // re,),
        in_specs=[
            _row_spec(re, hid),
            _row_spec(re, hid),
            _full_spec(hid, api),
            _full_spec(1, api),
            _full_spec(api, hid),
            _full_spec(1, hid),
        ],
        out_specs=[pl.BlockSpec((LANES, re * LANES), lambda i: (0, i)),
                   _row_spec(re, hid)],
        out_shape=[jax.ShapeDtypeStruct((LANES, e * LANES), f32),
                   jax.ShapeDtypeStruct((e, hid), jnp.bfloat16)],
    )(sa_mat, sb_mat, m2_w2, m2_b2.reshape(1, api), m4_w1[2 * api:],
      m4_b1.reshape(1, hid))


    n_chunks2 = e // K
    pairs = n_chunks2 // 2
    assert n_chunks2 % 2 == 0

    @functools.partial(
        pl.kernel,
        out_type=jax.ShapeDtypeStruct((LANES, n * LANES), f32),
        mesh=mesh,
        scratch_types=[
            pltpu.VMEM((LANES,), jnp.int32),
            pltpu.VMEM((K,), jnp.int32),
            pltpu.VMEM((K,), jnp.int32),
            pltpu.VMEM((K * LANES,), f32),
            pltpu.VMEM((K * LANES,), f32),
            pltpu.VMEM((tbl_rows * LANES,), f32),
            pltpu.SemaphoreType.DMA,
            pltpu.SemaphoreType.DMA,
            pltpu.SemaphoreType.DMA,
            pltpu.SemaphoreType.DMA,
        ],
    )
    def _sc2(skip_hbm, dst_hbm, iota_hbm, out_hbm, iv, didx0, didx1, rows0,
             rows1, tab_v, sd0, sd1, sr0, sr1):
        c = lax.axis_index("c")
        s = lax.axis_index("s")
        base = c * half
        pltpu.sync_copy(iota_hbm.at[pl.ds(0, LANES)], iv)
        tvec = half + iv[pl.ds(0, LANES)]

        def zr(rr, carry):
            tab_v[pl.ds(rr * LANES, LANES)] = jnp.zeros((LANES,), f32)
            return carry

        lax.fori_loop(0, tbl_rows, zr, 0)

        pltpu.async_copy(dst_hbm.at[pl.ds(0, K)], didx0, sd0)
        pltpu.async_copy(skip_hbm.at[s].at[pl.ds(0, K * LANES)], rows0, sr0)
        pltpu.async_copy(dst_hbm.at[pl.ds(K, K)], didx1, sd1)
        pltpu.async_copy(skip_hbm.at[s].at[pl.ds(K * LANES, K * LANES)],
                         rows1, sr1)

        def process(didx_v, rows_v):
            def grp(j, carry2):
                v = didx_v[pl.ds(j * LANES, LANES)]
                m = (v >= base) & (v < base + half)
                lv = jnp.where(m, v - base, tvec) * LANES
                for l in range(LANES):
                    p = lv[l]
                    q = (j * LANES + l) * LANES
                    tab_v[pl.ds(p, LANES)] = (
                        tab_v[pl.ds(p, LANES)] + rows_v[pl.ds(q, LANES)])
                return carry2

            lax.fori_loop(0, K // LANES, grp, 0)

        def pair(pi, carry):
            c0 = 2 * pi

            pltpu.make_async_copy(dst_hbm.at[pl.ds(0, K)], didx0, sd0).wait()
            pltpu.make_async_copy(skip_hbm.at[s].at[pl.ds(0, K * LANES)],
                                  rows0, sr0).wait()
            process(didx0, rows0)

            @pl.when(c0 + 2 < n_chunks2)
            def _():
                off = c0 + 2
                pltpu.async_copy(dst_hbm.at[pl.ds(off * K, K)], didx0, sd0)
                pltpu.async_copy(
                    skip_hbm.at[s].at[pl.ds(off * K * LANES, K * LANES)],
                    rows0, sr0)

            pltpu.make_async_copy(dst_hbm.at[pl.ds(0, K)], didx1, sd1).wait()
            pltpu.make_async_copy(skip_hbm.at[s].at[pl.ds(0, K * LANES)],
                                  rows1, sr1).wait()
            process(didx1, rows1)

            @pl.when(c0 + 3 < n_chunks2)
            def _():
                off = c0 + 3
                pltpu.async_copy(dst_hbm.at[pl.ds(off * K, K)], didx1, sd1)
                pltpu.async_copy(
                    skip_hbm.at[s].at[pl.ds(off * K * LANES, K * LANES)],
                    rows1, sr1)

            return carry

        lax.fori_loop(0, pairs, pair, 0)
        pltpu.sync_copy(tab_v.at[pl.ds(0, half * LANES)],
                        out_hbm.at[s].at[pl.ds(base * LANES, half * LANES)])

    feats0_g = _sc2(skip_g, dst, jnp.arange(LANES, dtype=jnp.int32))

    feats, c_mat, d_mat = pl.pallas_call(
        _tc3_body,
        grid=(n // rn,),
        in_specs=[
            pl.BlockSpec((LANES, rn * LANES), lambda i: (0, i)),
            _full_spec(api, hid),
            _full_spec(1, hid),
            _full_spec(hid, api),
            _full_spec(1, api),
            _full_spec(2 * api, hid),
        ],
        out_specs=[_row_spec(rn, api), _row_spec(rn, hid),
                   _row_spec(rn, hid)],
        out_shape=[jax.ShapeDtypeStruct((n, api), f32),
                   jax.ShapeDtypeStruct((n, hid), f32),
                   jax.ShapeDtypeStruct((n, hid), f32)],
    )(feats0_g, m3_w1, m3_b1.reshape(1, hid), m3_w2, m3_b2.reshape(1, api),
      m4_w1[:2 * api])

    ce_mat, de_mat = _gather2_k(_gather2)(c_mat, d_mat, src, dst)

    ew = pl.pallas_call(
        _tc4_body,
        grid=(e // re,),
        in_specs=[
            _row_spec(re, hid),
            _row_spec(re, hid),
            _row_spec(re, hid),
            _full_spec(hid, api),
            _full_spec(1, api),
            _full_spec(1, api),
            _full_spec(1, 1),
        ],
        out_specs=_row_spec(re, 1),
        out_shape=jax.ShapeDtypeStruct((e, 1), f32),
    )(ce_mat, de_mat, t_mat, m4_w2, m4_b2.reshape(1, api),
      fc_w.reshape(1, api), fc_b.reshape(1, 1))

    return (ew, feats)


# speedup vs baseline: 1.0250x; 1.0008x over previous
import functools

import jax
import jax.numpy as jnp
from jax import lax
from jax.experimental import pallas as pl
from jax.experimental.pallas import tpu as pltpu
import jax.experimental.pallas.tpu_sc as plsc

NC = 2
NS = 16
LANES = 16
K = 400
KG = 40


def _elu(x):
    return jnp.where(x > 0, x, jnp.exp(jnp.minimum(x, 0.0)) - 1.0)


def _dot(x, w):
    return jnp.dot(x.astype(jnp.bfloat16), w.astype(jnp.bfloat16),
                   preferred_element_type=jnp.float32)


def _tc1_body(x_ref, gf_ref, w1_ref, b1_ref, w2_ref, b2_ref, w21_ref,
              b21_ref, a_ref, b_ref):
    api = x_ref.shape[1]
    w1 = w1_ref[...]
    b1v = jnp.dot(gf_ref[...], w1[api:], preferred_element_type=jnp.float32) + b1_ref[...]
    h = _elu(_dot(x_ref[...], w1[:api]) + b1v)
    out = _elu(_dot(h, w2_ref[...]) + b2_ref[...])
    w21 = w21_ref[...]
    a_ref[...] = _dot(out, w21[:api]) + b21_ref[...]
    b_ref[...] = _dot(out, w21[api:])


def _tc2_body(sa_ref, sb_ref, w22_ref, b22_ref, w4e_ref, b4_ref, skip_ref,
              t_ref):
    h = _elu(sa_ref[...] + sb_ref[...])
    re = h.shape[0]
    sk = _elu(_dot(h, w22_ref[...]) + b22_ref[...])
    skip_ref[...] = jnp.transpose(sk.reshape(re, LANES, LANES),
                                  (1, 0, 2)).reshape(LANES, re * LANES)
    t_ref[...] = (_dot(sk, w4e_ref[...]) + b4_ref[...]).astype(jnp.bfloat16)


def _tc3_body(f0_ref, w31_ref, b31_ref, w32_ref, b32_ref, w4sd_ref,
              feats_ref, c_ref, d_ref):
    rn = f0_ref.shape[1] // LANES
    api = LANES * LANES
    f0 = jnp.transpose(f0_ref[...].reshape(LANES, rn, LANES),
                       (1, 0, 2)).reshape(rn, api)
    h = _elu(_dot(f0, w31_ref[...]) + b31_ref[...])
    f = _elu(_dot(h, w32_ref[...]) + b32_ref[...])
    feats_ref[...] = f
    w = w4sd_ref[...]
    c_ref[...] = _dot(f, w[:api])
    d_ref[...] = _dot(f, w[api:])


def _tc4_body(ce_ref, de_ref, t_ref, w42_ref, b42_ref, fcwt_ref, fcb_ref,
              ew_ref):
    h4 = _elu(ce_ref[...] + de_ref[...] + t_ref[...].astype(jnp.float32))
    ef3 = _elu(_dot(h4, w42_ref[...]) + b42_ref[...])
    v = jnp.sum(ef3 * fcwt_ref[...], axis=1, keepdims=True) + fcb_ref[...]
    ew_ref[...] = jax.nn.sigmoid(v)


def _row_spec(rows, cols):
    return pl.BlockSpec((rows, cols), lambda i: (i, 0))


def _full_spec(rows, cols):
    return pl.BlockSpec((rows, cols), lambda i: (0, 0))


def kernel(node_feats, graph_feats, m1_w1, m1_b1, m1_w2, m1_b2,
           m2_w1, m2_b1, m2_w2, m2_b2, m3_w1, m3_b1, m3_w2, m3_b2,
           m4_w1, m4_b1, m4_w2, m4_b2, fc_w, fc_b, edge_index):
    n, api = node_feats.shape
    hid = m1_w1.shape[1]
    e = edge_index.shape[1]
    mash = graph_feats.shape[0]
    f32 = jnp.float32
    bf16 = jnp.bfloat16

    src = edge_index[0]
    dst = edge_index[1]

    rn = 1000
    re = 1000
    assert n % rn == 0 and e % re == 0 and e % K == 0 and e % KG == 0 and api == LANES * LANES
    nw = NC * NS
    n_chunks = e // K
    half = n // NC
    tbl_rows = half + LANES

    a_mat, b_mat = pl.pallas_call(
        _tc1_body,
        grid=(n // rn,),
        in_specs=[
            _row_spec(rn, api),
            _full_spec(1, mash),
            _full_spec(api + mash, hid),
            _full_spec(1, hid),
            _full_spec(hid, api),
            _full_spec(1, api),
            _full_spec(2 * api, hid),
            _full_spec(1, hid),
        ],
        out_specs=[_row_spec(rn, hid), _row_spec(rn, hid)],
        out_shape=[jax.ShapeDtypeStruct((n, hid), f32),
                   jax.ShapeDtypeStruct((n, hid), f32)],
    )(node_feats, graph_feats.reshape(1, mash), m1_w1, m1_b1.reshape(1, hid),
      m1_w2, m1_b2.reshape(1, api), m2_w1, m2_b1.reshape(1, hid))

    mesh = plsc.VectorSubcoreMesh(core_axis_name="c", subcore_axis_name="s",
                                  num_cores=NC, num_subcores=NS)
    ng_chunks = e // KG
    steps_g = (ng_chunks + nw - 1) // nw

    def _gather2(a_hbm, b_hbm, src_hbm, dst_hbm, sa_hbm, sb_hbm,
                 sidx0, didx0, sidx1, didx1, ra0, rb0, ra1, rb1,
                 sa0, sb0, sa1, sb1):
        wid = lax.axis_index("s") * NC + lax.axis_index("c")

        def issue(chunk, sidx_v, didx_v, rows_a, rows_b, sema, semb):
            off = chunk * KG
            pltpu.sync_copy(src_hbm.at[pl.ds(off, KG)], sidx_v)
            pltpu.sync_copy(dst_hbm.at[pl.ds(off, KG)], didx_v)
            pltpu.async_copy(a_hbm.at[sidx_v], rows_a, sema)
            pltpu.async_copy(b_hbm.at[didx_v], rows_b, semb)

        def drain_write(chunk, rows_a, rows_b, sema, semb):
            off = chunk * KG
            pltpu.make_async_copy(a_hbm.at[pl.ds(0, KG)], rows_a, sema).wait()
            pltpu.make_async_copy(b_hbm.at[pl.ds(0, KG)], rows_b, semb).wait()
            pltpu.sync_copy(rows_a, sa_hbm.at[pl.ds(off, KG)])
            pltpu.sync_copy(rows_b, sb_hbm.at[pl.ds(off, KG)])

        ch0 = wid

        @pl.when(ch0 < ng_chunks)
        def _():
            issue(ch0, sidx0, didx0, ra0, rb0, sa0, sb0)

        g_pairs = (steps_g + 1) // 2

        def pair(pi, carry):
            it0 = 2 * pi
            cA = wid + it0 * nw
            cB = cA + nw
            cA2 = cB + nw

            @pl.when(cB < ng_chunks)
            def _():
                issue(cB, sidx1, didx1, ra1, rb1, sa1, sb1)

            @pl.when(cA < ng_chunks)
            def _():
                drain_write(cA, ra0, rb0, sa0, sb0)

            @pl.when(cA2 < ng_chunks)
            def _():
                issue(cA2, sidx0, didx0, ra0, rb0, sa0, sb0)

            @pl.when(cB < ng_chunks)
            def _():
                drain_write(cB, ra1, rb1, sa1, sb1)

            return carry

        lax.fori_loop(0, g_pairs, pair, 0)

    _gather2_k = functools.partial(
        pl.kernel,
        out_type=(jax.ShapeDtypeStruct((e, hid), f32),
                  jax.ShapeDtypeStruct((e, hid), f32)),
        mesh=mesh,
        scratch_types=[
            pltpu.VMEM((KG,), jnp.int32),
            pltpu.VMEM((KG,), jnp.int32),
            pltpu.VMEM((KG,), jnp.int32),
            pltpu.VMEM((KG,), jnp.int32),
            pltpu.VMEM((KG, hid), f32),
            pltpu.VMEM((KG, hid), f32),
            pltpu.VMEM((KG, hid), f32),
            pltpu.VMEM((KG, hid), f32),
            pltpu.SemaphoreType.DMA,
            pltpu.SemaphoreType.DMA,
            pltpu.SemaphoreType.DMA,
            pltpu.SemaphoreType.DMA,
        ],
    )
    sa_mat, sb_mat = _gather2_k(_gather2)(a_mat, b_mat, src, dst)

    skip_g, t_mat = pl.pallas_call(
        _tc2_body,
        grid=(e // re,),
        in_specs=[
            _row_spec(re, hid),
            _row_spec(re, hid),
            _full_spec(hid, api),
            _full_spec(1, api),
            _full_spec(api, hid),
            _full_spec(1, hid),
        ],
        out_specs=[pl.BlockSpec((LANES, re * LANES), lambda i: (0, i)),
                   _row_spec(re, hid)],
        out_shape=[jax.ShapeDtypeStruct((LANES, e * LANES), f32),
                   jax.ShapeDtypeStruct((e, hid), jnp.bfloat16)],
    )(sa_mat, sb_mat, m2_w2, m2_b2.reshape(1, api), m4_w1[2 * api:],
      m4_b1.reshape(1, hid))


    n_chunks2 = e // K
    pairs = n_chunks2 // 2
    assert n_chunks2 % 2 == 0

    @functools.partial(
        pl.kernel,
        out_type=jax.ShapeDtypeStruct((LANES, n * LANES), f32),
        mesh=mesh,
        scratch_types=[
            pltpu.VMEM((LANES,), jnp.int32),
            pltpu.VMEM((K,), jnp.int32),
            pltpu.VMEM((K,), jnp.int32),
            pltpu.VMEM((K * LANES,), f32),
            pltpu.VMEM((K * LANES,), f32),
            pltpu.VMEM((tbl_rows * LANES,), f32),
            pltpu.SemaphoreType.DMA,
            pltpu.SemaphoreType.DMA,
            pltpu.SemaphoreType.DMA,
            pltpu.SemaphoreType.DMA,
        ],
    )
    def _sc2(skip_hbm, dst_hbm, iota_hbm, out_hbm, iv, didx0, didx1, rows0,
             rows1, tab_v, sd0, sd1, sr0, sr1):
        c = lax.axis_index("c")
        s = lax.axis_index("s")
        base = c * half
        pltpu.sync_copy(iota_hbm.at[pl.ds(0, LANES)], iv)
        tvec = half + iv[pl.ds(0, LANES)]

        def zr(rr, carry):
            tab_v[pl.ds(rr * LANES, LANES)] = jnp.zeros((LANES,), f32)
            return carry

        lax.fori_loop(0, tbl_rows, zr, 0)

        pltpu.async_copy(dst_hbm.at[pl.ds(0, K)], didx0, sd0)
        pltpu.async_copy(skip_hbm.at[s].at[pl.ds(0, K * LANES)], rows0, sr0)
        pltpu.async_copy(dst_hbm.at[pl.ds(K, K)], didx1, sd1)
        pltpu.async_copy(skip_hbm.at[s].at[pl.ds(K * LANES, K * LANES)],
                         rows1, sr1)

        def process(didx_v, rows_v):
            def grp(j, carry2):
                v = didx_v[pl.ds(j * LANES, LANES)]
                m = (v >= base) & (v < base + half)
                lv = jnp.where(m, v - base, tvec) * LANES
                for l in range(LANES):
                    p = lv[l]
                    q = (j * LANES + l) * LANES
                    tab_v[pl.ds(p, LANES)] = (
                        tab_v[pl.ds(p, LANES)] + rows_v[pl.ds(q, LANES)])
                return carry2

            lax.fori_loop(0, K // LANES, grp, 0)

        def pair(pi, carry):
            c0 = 2 * pi

            pltpu.make_async_copy(dst_hbm.at[pl.ds(0, K)], didx0, sd0).wait()
            pltpu.make_async_copy(skip_hbm.at[s].at[pl.ds(0, K * LANES)],
                                  rows0, sr0).wait()
            process(didx0, rows0)

            @pl.when(c0 + 2 < n_chunks2)
            def _():
                off = c0 + 2
                pltpu.async_copy(dst_hbm.at[pl.ds(off * K, K)], didx0, sd0)
                pltpu.async_copy(
                    skip_hbm.at[s].at[pl.ds(off * K * LANES, K * LANES)],
                    rows0, sr0)

            pltpu.make_async_copy(dst_hbm.at[pl.ds(0, K)], didx1, sd1).wait()
            pltpu.make_async_copy(skip_hbm.at[s].at[pl.ds(0, K * LANES)],
                                  rows1, sr1).wait()
            process(didx1, rows1)

            @pl.when(c0 + 3 < n_chunks2)
            def _():
                off = c0 + 3
                pltpu.async_copy(dst_hbm.at[pl.ds(off * K, K)], didx1, sd1)
                pltpu.async_copy(
                    skip_hbm.at[s].at[pl.ds(off * K * LANES, K * LANES)],
                    rows1, sr1)

            return carry

        lax.fori_loop(0, pairs, pair, 0)
        pltpu.sync_copy(tab_v.at[pl.ds(0, half * LANES)],
                        out_hbm.at[s].at[pl.ds(base * LANES, half * LANES)])

    feats0_g = _sc2(skip_g, dst, jnp.arange(LANES, dtype=jnp.int32))

    feats, c_mat, d_mat = pl.pallas_call(
        _tc3_body,
        grid=(n // rn,),
        in_specs=[
            pl.BlockSpec((LANES, rn * LANES), lambda i: (0, i)),
            _full_spec(api, hid),
            _full_spec(1, hid),
            _full_spec(hid, api),
            _full_spec(1, api),
            _full_spec(2 * api, hid),
        ],
        out_specs=[_row_spec(rn, api), _row_spec(rn, hid),
                   _row_spec(rn, hid)],
        out_shape=[jax.ShapeDtypeStruct((n, api), f32),
                   jax.ShapeDtypeStruct((n, hid), f32),
                   jax.ShapeDtypeStruct((n, hid), f32)],
    )(feats0_g, m3_w1, m3_b1.reshape(1, hid), m3_w2, m3_b2.reshape(1, api),
      m4_w1[:2 * api])

    ce_mat, de_mat = _gather2_k(_gather2)(c_mat, d_mat, src, dst)

    ew = pl.pallas_call(
        _tc4_body,
        grid=(e // re,),
        in_specs=[
            _row_spec(re, hid),
            _row_spec(re, hid),
            _row_spec(re, hid),
            _full_spec(hid, api),
            _full_spec(1, api),
            _full_spec(1, api),
            _full_spec(1, 1),
        ],
        out_specs=_row_spec(re, 1),
        out_shape=jax.ShapeDtypeStruct((e, 1), f32),
    )(ce_mat, de_mat, t_mat, m4_w2, m4_b2.reshape(1, api),
      fc_w.reshape(1, api), fc_b.reshape(1, 1))

    return (ew, feats)
